# baseline (speedup 1.0000x reference)
import functools

import jax
import jax.numpy as jnp
from jax import lax
from jax.experimental import pallas as pl


def _body(state_ref, rp_ref, mem_ref, tab_ref, out_ref, idx_ref, fit_ref, *,
          nb, heads, inv_b):
    i = pl.program_id(0)
    proj = lax.dot_general(
        state_ref[...], rp_ref[...], (((1,), (0,)), ((), ())),
        preferred_element_type=jnp.float32,
        precision=lax.Precision.DEFAULT)
    sims = lax.dot_general(
        mem_ref[...], proj, (((1,), (1,)), ((), ())),
        preferred_element_type=jnp.float32,
        precision=lax.Precision.DEFAULT)
    slots = 64
    bt = sims.shape[1]
    ngrp = heads // slots
    sims_r = sims.reshape(ngrp, slots, bt)
    vm = sims_r[0]
    vg = jnp.zeros((slots, bt), jnp.int32)
    for g in range(1, ngrp):
        c = sims_r[g]
        gt = c > vm
        vm = jnp.where(gt, c, vm)
        vg = jnp.where(gt, g, vg)
    vh = vg * slots + lax.broadcasted_iota(jnp.int32, (slots, bt), 0)
    m = jnp.max(vm, axis=0, keepdims=True)
    idx = jnp.min(jnp.where(vm == m, vh, heads), axis=0,
                  keepdims=True)
    idx_ref[...] = idx

    onehot = (lax.broadcasted_iota(jnp.int32, (heads, bt), 0)
              == idx).astype(jnp.bfloat16)
    out_ref[...] = lax.dot_general(
        onehot, tab_ref[...].astype(jnp.bfloat16),
        (((0,), (0,)), ((), ())),
        preferred_element_type=jnp.float32)

    @pl.when(i == 0)
    def _():
        fit_ref[...] = jnp.zeros_like(fit_ref)

    fit_ref[...] += jnp.sum(m, axis=1, keepdims=True)

    @pl.when(i == nb - 1)
    def _():
        fit_ref[...] = fit_ref[...] * inv_b


def kernel(state, random_projection, memories, logits_table):
    b, in_dim = state.shape
    proj_dim = random_projection.shape[1]
    heads = memories.shape[0]
    act_dim = logits_table.shape[1]
    bt = 1024
    nb = b // bt

    out_logits, idx, fit = pl.pallas_call(
        functools.partial(_body, nb=nb, heads=heads, inv_b=1.0 / b),
        grid=(nb,),
        in_specs=[
            pl.BlockSpec((bt, in_dim), lambda i: (i, 0)),
            pl.BlockSpec((in_dim, proj_dim), lambda i: (0, 0)),
            pl.BlockSpec((heads, proj_dim), lambda i: (0, 0)),
            pl.BlockSpec((heads, act_dim), lambda i: (0, 0)),
        ],
        out_specs=[
            pl.BlockSpec((bt, act_dim), lambda i: (i, 0)),
            pl.BlockSpec((1, bt), lambda i: (0, i)),
            pl.BlockSpec((1, 1), lambda i: (0, 0)),
        ],
        out_shape=[
            jax.ShapeDtypeStruct((b, act_dim), jnp.float32),
            jax.ShapeDtypeStruct((1, b), jnp.int32),
            jax.ShapeDtypeStruct((1, 1), jnp.float32),
        ],
    )(state, random_projection, memories, logits_table)

    return out_logits, fit[0, 0]


# speedup vs baseline: 1.7121x; 1.7121x over previous
import functools

import jax
import jax.numpy as jnp
from jax import lax
from jax.experimental import pallas as pl


def _body(state_ref, rp_ref, mem_ref, tab_ref, out_ref, fit_ref, *,
          nb, heads, inv_b, act_dim, sub):
    i = pl.program_id(0)
    proj = lax.dot_general(
        state_ref[...], rp_ref[...], (((1,), (0,)), ((), ())),
        preferred_element_type=jnp.float32,
        precision=lax.Precision.DEFAULT)
    sims = lax.dot_general(
        mem_ref[...], proj, (((1,), (1,)), ((), ())),
        preferred_element_type=jnp.float32,
        precision=lax.Precision.DEFAULT)
    slots = 64
    bt = sims.shape[1]
    ngrp = heads // slots
    sims_r = sims.reshape(ngrp, slots, bt)
    vm = sims_r[0]
    vg = jnp.zeros((slots, bt), jnp.int32)
    for g in range(1, ngrp):
        c = sims_r[g]
        gt = c > vm
        vm = jnp.where(gt, c, vm)
        vg = jnp.where(gt, g, vg)
    vh = vg * slots + lax.broadcasted_iota(jnp.int32, (slots, bt), 0)
    m = jnp.max(vm, axis=0, keepdims=True)
    idx = jnp.min(jnp.where(vm == m, vh, heads), axis=0,
                  keepdims=True)

    grp = heads // sub
    idx_c = idx.reshape(bt, 1)
    g_c = idx_c // sub
    s_c = idx_c % sub
    onehot_g = (lax.broadcasted_iota(jnp.int32, (bt, grp), 1)
                == g_c).astype(jnp.bfloat16)
    cand = lax.dot_general(
        onehot_g, tab_ref[...].astype(jnp.bfloat16),
        (((1,), (0,)), ((), ())),
        preferred_element_type=jnp.float32)
    lane_s = lax.broadcasted_iota(jnp.int32, (bt, sub * act_dim), 1) // act_dim
    picked = jnp.where(lane_s == s_c, cand, 0.0)
    out_ref[...] = picked.reshape(bt, sub, act_dim).sum(axis=1)

    @pl.when(i == 0)
    def _():
        fit_ref[...] = jnp.zeros_like(fit_ref)

    fit_ref[...] += jnp.sum(m, axis=1, keepdims=True)

    @pl.when(i == nb - 1)
    def _():
        fit_ref[...] = fit_ref[...] * inv_b


def kernel(state, random_projection, memories, logits_table):
    b, in_dim = state.shape
    proj_dim = random_projection.shape[1]
    heads = memories.shape[0]
    act_dim = logits_table.shape[1]
    sub = 16
    bt = 1024
    nb = b // bt
    tabr = logits_table.reshape(heads // sub, sub * act_dim)

    out_logits, fit = pl.pallas_call(
        functools.partial(_body, nb=nb, heads=heads, inv_b=1.0 / b,
                          act_dim=act_dim, sub=sub),
        grid=(nb,),
        in_specs=[
            pl.BlockSpec((bt, in_dim), lambda i: (i, 0)),
            pl.BlockSpec((in_dim, proj_dim), lambda i: (0, 0)),
            pl.BlockSpec((heads, proj_dim), lambda i: (0, 0)),
            pl.BlockSpec(tabr.shape, lambda i: (0, 0)),
        ],
        out_specs=[
            pl.BlockSpec((bt, act_dim), lambda i: (i, 0)),
            pl.BlockSpec((1, 1), lambda i: (0, 0)),
        ],
        out_shape=[
            jax.ShapeDtypeStruct((b, act_dim), jnp.float32),
            jax.ShapeDtypeStruct((1, 1), jnp.float32),
        ],
    )(state, random_projection, memories, tabr)

    return out_logits, fit[0, 0]
